# baseline (speedup 1.0000x reference)
import functools

import jax
import jax.numpy as jnp
from jax import lax
from jax.experimental import pallas as pl
from jax.experimental.pallas import tpu as pltpu
from jax.experimental.pallas import tpu_sc as plsc

_N = 10000
_E = 320000
_H = 128
_EF = 16
_G = 64

_BN = 2000
_BE = 4000

_NC = 2
_NS = 16
_NW = _NC * _NS
_EPW = _E // _NW
_K = 80
_NCHUNK = _EPW // _K
_WPT = 624
_ZB = 48
_TAIL = _N - _NS * _WPT


def _silu(x):
    return x * jax.nn.sigmoid(x)


def _dual_res_body(x_ref, w1, b1, w2, b2, w3, b3, w4, b4, src_ref, dst_ref):
    x = x_ref[...]
    h = _silu(jnp.dot(x, w1[...], preferred_element_type=jnp.float32) + b1[...])
    h = _silu(jnp.dot(h, w2[...], preferred_element_type=jnp.float32) + b2[...])
    src_ref[...] = x + h
    h = _silu(jnp.dot(x, w3[...], preferred_element_type=jnp.float32) + b3[...])
    h = _silu(jnp.dot(h, w4[...], preferred_element_type=jnp.float32) + b4[...])
    dst_ref[...] = x + h


def _dual_residual(x, emb0, emb1):
    ws = []
    for (w, b) in emb0 + emb1:
        ws.append(w)
        ws.append(b.reshape(1, _H))
    wspec = pl.BlockSpec((_H, _H), lambda i: (0, 0))
    bspec = pl.BlockSpec((1, _H), lambda i: (0, 0))
    return pl.pallas_call(
        _dual_res_body,
        grid=(_N // _BN,),
        in_specs=[pl.BlockSpec((_BN, _H), lambda i: (i, 0))]
        + [wspec, bspec, wspec, bspec, wspec, bspec, wspec, bspec],
        out_specs=[pl.BlockSpec((_BN, _H), lambda i: (i, 0))] * 2,
        out_shape=[jax.ShapeDtypeStruct((_N, _H), jnp.float32)] * 2,
    )(x, *ws)


def _filter_body(ef_ref, sm_ref, el_ref, f_ref):
    ef = ef_ref[...]
    sm = sm_ref[...]
    f_ref[...] = jnp.dot(ef, el_ref[...], preferred_element_type=jnp.float32) * sm


def _filters(ef, conv_smooth, el):
    return pl.pallas_call(
        _filter_body,
        grid=(_E // _BE,),
        in_specs=[
            pl.BlockSpec((_BE, _EF), lambda i: (i, 0)),
            pl.BlockSpec((_BE, 1), lambda i: (i, 0)),
            pl.BlockSpec((_EF, _H), lambda i: (0, 0)),
        ],
        out_specs=pl.BlockSpec((_BE, _H), lambda i: (i, 0)),
        out_shape=jax.ShapeDtypeStruct((_E, _H), jnp.float32),
    )(ef, conv_smooth, el)


def _sc_body(dst_hbm, filt_hbm, col_hbm, row_hbm, out_hbm,
             colv_a, rowv_a, colv_b, rowv_b, rows_a, rows_b, filt_a, filt_b,
             zbuf, acc, semi_a, semi_b, semf_a, semf_b, semg_a, semg_b, sems_a):
    cid = lax.axis_index("c")
    sid = lax.axis_index("s")

    def _zrow(k, _):
        for j in range(_H // 16):
            zbuf[k, pl.ds(j * 16, 16)] = jnp.zeros((16,), jnp.float32)
        return 0

    lax.fori_loop(0, _ZB, _zrow, 0)
    for r in range(_WPT // _ZB):
        pltpu.sync_copy(zbuf, acc.at[pl.ds(sid * _WPT + r * _ZB, _ZB)])

    @pl.when(sid == 0)
    def _():
        pltpu.sync_copy(zbuf.at[pl.ds(0, _TAIL)], acc.at[pl.ds(_NS * _WPT, _TAIL)])

    plsc.subcore_barrier()

    wid = cid * _NS + sid
    tile_base = wid * _EPW

    def idx_copy(c, colv, rowv, semi):
        base = tile_base + c * _K
        pltpu.async_copy(col_hbm.at[pl.ds(base, _K)], colv, semi)
        pltpu.async_copy(row_hbm.at[pl.ds(base, _K)], rowv, semi)

    def idx_wait(colv, rowv, semi):
        pltpu.make_async_copy(col_hbm.at[pl.ds(tile_base, _K)], colv, semi).wait()
        pltpu.make_async_copy(row_hbm.at[pl.ds(tile_base, _K)], rowv, semi).wait()

    def filt_copy(c, filt, semf):
        base = tile_base + c * _K
        pltpu.async_copy(filt_hbm.at[pl.ds(base, _K)], filt, semf)

    def filt_wait(filt, semf):
        pltpu.make_async_copy(filt_hbm.at[pl.ds(tile_base, _K)], filt, semf).wait()

    def gather_issue(colv, rows, semg):
        pltpu.async_copy(dst_hbm.at[colv], rows, semg)

    def gather_wait(colv, rows, semg):
        pltpu.make_async_copy(dst_hbm.at[colv], rows, semg).wait()

    def mul(rows, filt):
        def _mul(k, _):
            for j in range(_H // 16):
                s = pl.ds(j * 16, 16)
                rows[k, s] = rows[k, s] * filt[k, s]
            return 0

        lax.fori_loop(0, _K, _mul, 0)

    idx_copy(0, colv_a, rowv_a, semi_a)
    filt_copy(0, filt_a, semf_a)
    idx_copy(1, colv_b, rowv_b, semi_b)
    filt_copy(1, filt_b, semf_b)
    idx_wait(colv_a, rowv_a, semi_a)
    gather_issue(colv_a, rows_a, semg_a)

    def _pair(i, _):
        c0 = 2 * i
        idx_wait(colv_b, rowv_b, semi_b)
        gather_issue(colv_b, rows_b, semg_b)

        gather_wait(colv_a, rows_a, semg_a)
        filt_wait(filt_a, semf_a)
        mul(rows_a, filt_a)
        pltpu.async_copy(rows_a, acc.at[rowv_a], sems_a, add=True)

        gather_wait(colv_b, rows_b, semg_b)
        filt_wait(filt_b, semf_b)
        mul(rows_b, filt_b)
        pltpu.make_async_copy(rows_a, acc.at[rowv_a], sems_a).wait()
        pltpu.sync_copy(rows_b, acc.at[rowv_b], add=True)

        @pl.when(c0 + 2 < _NCHUNK)
        def _():
            idx_copy(c0 + 2, colv_a, rowv_a, semi_a)
            filt_copy(c0 + 2, filt_a, semf_a)

        @pl.when(c0 + 3 < _NCHUNK)
        def _():
            idx_copy(c0 + 3, colv_b, rowv_b, semi_b)
            filt_copy(c0 + 3, filt_b, semf_b)

        @pl.when(c0 + 2 < _NCHUNK)
        def _():
            idx_wait(colv_a, rowv_a, semi_a)
            gather_issue(colv_a, rows_a, semg_a)

        return 0

    lax.fori_loop(0, _NCHUNK // 2, _pair, 0)
    gather_wait(colv_a, rows_a, semg_a)
    filt_wait(filt_a, semf_a)
    mul(rows_a, filt_a)
    pltpu.sync_copy(rows_a, acc.at[rowv_a], add=True)
    plsc.subcore_barrier()

    pltpu.sync_copy(acc.at[pl.ds(sid * _WPT, _WPT)],
                    out_hbm.at[cid, pl.ds(sid * _WPT, _WPT)])

    @pl.when(sid == 0)
    def _():
        pltpu.sync_copy(acc.at[pl.ds(_NS * _WPT, _TAIL)],
                        out_hbm.at[cid, pl.ds(_NS * _WPT, _TAIL)])


def _sc_gather_scatter(dst, filt, col, row):
    mesh = plsc.VectorSubcoreMesh(core_axis_name="c", subcore_axis_name="s")
    f = pl.kernel(
        _sc_body,
        mesh=mesh,
        out_type=jax.ShapeDtypeStruct((_NC, _N, _H), jnp.float32),
        scratch_types=[
            pltpu.VMEM((_K,), jnp.int32),
            pltpu.VMEM((_K,), jnp.int32),
            pltpu.VMEM((_K,), jnp.int32),
            pltpu.VMEM((_K,), jnp.int32),
            pltpu.VMEM((_K, _H), jnp.float32),
            pltpu.VMEM((_K, _H), jnp.float32),
            pltpu.VMEM((_K, _H), jnp.float32),
            pltpu.VMEM((_K, _H), jnp.float32),
            pltpu.VMEM((_ZB, _H), jnp.float32),
            pltpu.VMEM_SHARED((_N, _H), jnp.float32),
            pltpu.SemaphoreType.DMA,
            pltpu.SemaphoreType.DMA,
            pltpu.SemaphoreType.DMA,
            pltpu.SemaphoreType.DMA,
            pltpu.SemaphoreType.DMA,
            pltpu.SemaphoreType.DMA,
            pltpu.SemaphoreType.DMA,
        ],
    )
    return f(dst, filt, col, row)


def _res_block(v, w_a, b_a, w_b, b_b):
    h = _silu(jnp.dot(v, w_a[...], preferred_element_type=jnp.float32) + b_a[...])
    h = _silu(jnp.dot(h, w_b[...], preferred_element_type=jnp.float32) + b_b[...])
    return v + h


def _post_chain(p0_ref, p1_ref, c_ref, src_ref, xin_ref, dw):
    conv = (p0_ref[...] + p1_ref[...]) * c_ref[0, 0]
    conv = _res_block(conv, *dw[0:4])
    conv = _res_block(conv, *dw[4:8])
    out = src_ref[...] * conv
    out = _res_block(out, *dw[8:12])
    out = _res_block(out, *dw[12:16])
    return out + xin_ref[...]


def _post_dual_body(p0_ref, p1_ref, c_ref, src_ref, xin_ref, *refs):
    dw = refs[0:16]
    aw = refs[16:24]
    x2_ref, src2_ref, dst2_ref = refs[24:27]
    x2 = _post_chain(p0_ref, p1_ref, c_ref, src_ref, xin_ref, dw)
    x2_ref[...] = x2
    h = _silu(jnp.dot(x2, aw[0][...], preferred_element_type=jnp.float32) + aw[1][...])
    h = _silu(jnp.dot(h, aw[2][...], preferred_element_type=jnp.float32) + aw[3][...])
    src2_ref[...] = x2 + h
    h = _silu(jnp.dot(x2, aw[4][...], preferred_element_type=jnp.float32) + aw[5][...])
    h = _silu(jnp.dot(h, aw[6][...], preferred_element_type=jnp.float32) + aw[7][...])
    dst2_ref[...] = x2 + h


def _post_dual(p0, p1, c, src, xin, conv_mlp, out_mlp, emb0, emb1):
    ws = []
    for res in conv_mlp + out_mlp:
        for (w, b) in res:
            ws.append(w)
            ws.append(b.reshape(1, _H))
    for (w, b) in emb0 + emb1:
        ws.append(w)
        ws.append(b.reshape(1, _H))
    blk = pl.BlockSpec((_BN, _H), lambda i: (i, 0))
    wspec = pl.BlockSpec((_H, _H), lambda i: (0, 0))
    bspec = pl.BlockSpec((1, _H), lambda i: (0, 0))
    return pl.pallas_call(
        _post_dual_body,
        grid=(_N // _BN,),
        in_specs=[blk, blk, pl.BlockSpec((1, 1), lambda i: (0, 0)), blk, blk]
        + [wspec, bspec] * 12,
        out_specs=[blk] * 3,
        out_shape=[jax.ShapeDtypeStruct((_N, _H), jnp.float32)] * 3,
    )(p0, p1, c.reshape(1, 1), src, xin, *ws)


def _post_pool_body(p0_ref, p1_ref, c_ref, src_ref, xin_ref, bi_ref, *refs):
    dw = refs[0:16]
    pw = refs[16:28]
    g_ref = refs[28]
    i = pl.program_id(0)
    out = _post_chain(p0_ref, p1_ref, c_ref, src_ref, xin_ref, dw)

    @pl.when(i == 0)
    def _():
        g_ref[...] = jnp.zeros((_G, _H), jnp.float32)

    oh = (bi_ref[0] == lax.broadcasted_iota(jnp.int32, (_G, _BN), 0)).astype(jnp.float32)
    g_ref[...] += jnp.dot(oh, out, preferred_element_type=jnp.float32)

    @pl.when(i == _N // _BN - 1)
    def _():
        g = g_ref[...]
        g = _res_block(g, *pw[0:4])
        g = _res_block(g, *pw[4:8])
        g = _res_block(g, *pw[8:12])
        g_ref[...] = g


def _post_pool(p0, p1, c, src, xin, batch_index, conv_mlp, out_mlp, pool_params):
    ws = []
    for res in conv_mlp + out_mlp:
        for (w, b) in res:
            ws.append(w)
            ws.append(b.reshape(1, _H))
    for res in pool_params:
        for (w, b) in res:
            ws.append(w)
            ws.append(b.reshape(1, _H))
    bi3 = batch_index.reshape(_N // _BN, 1, _BN)
    blk = pl.BlockSpec((_BN, _H), lambda i: (i, 0))
    wspec = pl.BlockSpec((_H, _H), lambda i: (0, 0))
    bspec = pl.BlockSpec((1, _H), lambda i: (0, 0))
    return pl.pallas_call(
        _post_pool_body,
        grid=(_N // _BN,),
        in_specs=[blk, blk, pl.BlockSpec((1, 1), lambda i: (0, 0)), blk, blk,
                  pl.BlockSpec((1, 1, _BN), lambda i: (i, 0, 0))]
        + [wspec, bspec] * 14,
        out_specs=pl.BlockSpec((_G, _H), lambda i: (0, 0)),
        out_shape=jax.ShapeDtypeStruct((_G, _H), jnp.float32),
    )(p0, p1, c.reshape(1, 1), src, xin, bi3, *ws)


def kernel(scalar, ef, edge_index, C, conv_smooth, batch_index, params):
    row = edge_index[0]
    col = edge_index[1]
    p1, p2 = params["convs"]
    f1 = _filters(ef, conv_smooth, p1["e_lin"])
    src1, dst1 = _dual_residual(scalar, p1["emb0"], p1["emb1"])
    part1 = _sc_gather_scatter(dst1, f1, col, row)
    f2 = _filters(ef, conv_smooth, p2["e_lin"])
    x2, src2, dst2 = _post_dual(part1[0], part1[1], C, src1, scalar,
                                p1["conv_mlp"], p1["out_mlp"],
                                p2["emb0"], p2["emb1"])
    part2 = _sc_gather_scatter(dst2, f2, col, row)
    return _post_pool(part2[0], part2[1], C, src2, x2, batch_index,
                      p2["conv_mlp"], p2["out_mlp"], params["pool"])


# speedup vs baseline: 1.0320x; 1.0320x over previous
import functools

import jax
import jax.numpy as jnp
from jax import lax
from jax.experimental import pallas as pl
from jax.experimental.pallas import tpu as pltpu
from jax.experimental.pallas import tpu_sc as plsc

_N = 10000
_E = 320000
_H = 128
_EF = 16
_G = 64

_BN = 2000
_BE = 4000

_NC = 2
_NS = 16
_NW = _NC * _NS
_EPW = _E // _NW
_K = 80
_NCHUNK = _EPW // _K
_WPT = 624
_ZB = 48
_TAIL = _N - _NS * _WPT


def _silu(x):
    return x * jax.nn.sigmoid(x)


def _dual_res_body(x_ref, w1, b1, w2, b2, w3, b3, w4, b4, src_ref, dst_ref):
    x = x_ref[...]
    h = _silu(jnp.dot(x, w1[...], preferred_element_type=jnp.float32) + b1[...])
    h = _silu(jnp.dot(h, w2[...], preferred_element_type=jnp.float32) + b2[...])
    src_ref[...] = x + h
    h = _silu(jnp.dot(x, w3[...], preferred_element_type=jnp.float32) + b3[...])
    h = _silu(jnp.dot(h, w4[...], preferred_element_type=jnp.float32) + b4[...])
    dst_ref[...] = x + h


def _dual_residual(x, emb0, emb1):
    ws = []
    for (w, b) in emb0 + emb1:
        ws.append(w)
        ws.append(b.reshape(1, _H))
    wspec = pl.BlockSpec((_H, _H), lambda i: (0, 0))
    bspec = pl.BlockSpec((1, _H), lambda i: (0, 0))
    return pl.pallas_call(
        _dual_res_body,
        grid=(_N // _BN,),
        in_specs=[pl.BlockSpec((_BN, _H), lambda i: (i, 0))]
        + [wspec, bspec, wspec, bspec, wspec, bspec, wspec, bspec],
        out_specs=[pl.BlockSpec((_BN, _H), lambda i: (i, 0))] * 2,
        out_shape=[jax.ShapeDtypeStruct((_N, _H), jnp.float32)] * 2,
    )(x, *ws)


def _filter_body(ef_ref, sm_ref, el_ref, f_ref):
    ef = ef_ref[...]
    sm = sm_ref[...]
    f_ref[...] = jnp.dot(ef, el_ref[...], preferred_element_type=jnp.float32) * sm


def _filters(ef, conv_smooth, el):
    return pl.pallas_call(
        _filter_body,
        grid=(_E // _BE,),
        in_specs=[
            pl.BlockSpec((_BE, _EF), lambda i: (i, 0)),
            pl.BlockSpec((_BE, 1), lambda i: (i, 0)),
            pl.BlockSpec((_EF, _H), lambda i: (0, 0)),
        ],
        out_specs=pl.BlockSpec((_BE, _H), lambda i: (i, 0)),
        out_shape=jax.ShapeDtypeStruct((_E, _H), jnp.float32),
    )(ef, conv_smooth, el)


def _sc_body(dst_hbm, filt_hbm, col_hbm, row_hbm, out_hbm,
             colv_a, rowv_a, colv_b, rowv_b, rows_a, rows_b, filt_a, filt_b,
             zbuf, acc, semi_a, semi_b, semf_a, semf_b, semg_a, semg_b):
    cid = lax.axis_index("c")
    sid = lax.axis_index("s")

    def _zrow(k, _):
        for j in range(_H // 16):
            zbuf[k, pl.ds(j * 16, 16)] = jnp.zeros((16,), jnp.float32)
        return 0

    lax.fori_loop(0, _ZB, _zrow, 0)
    for r in range(_WPT // _ZB):
        pltpu.sync_copy(zbuf, acc.at[pl.ds(sid * _WPT + r * _ZB, _ZB)])

    @pl.when(sid == 0)
    def _():
        pltpu.sync_copy(zbuf.at[pl.ds(0, _TAIL)], acc.at[pl.ds(_NS * _WPT, _TAIL)])

    plsc.subcore_barrier()

    wid = cid * _NS + sid
    tile_base = wid * _EPW

    def idx_copy(c, colv, rowv, semi):
        base = tile_base + c * _K
        pltpu.async_copy(col_hbm.at[pl.ds(base, _K)], colv, semi)
        pltpu.async_copy(row_hbm.at[pl.ds(base, _K)], rowv, semi)

    def idx_wait(colv, rowv, semi):
        pltpu.make_async_copy(col_hbm.at[pl.ds(tile_base, _K)], colv, semi).wait()
        pltpu.make_async_copy(row_hbm.at[pl.ds(tile_base, _K)], rowv, semi).wait()

    def filt_copy(c, filt, semf):
        base = tile_base + c * _K
        pltpu.async_copy(filt_hbm.at[pl.ds(base, _K)], filt, semf)

    def filt_wait(filt, semf):
        pltpu.make_async_copy(filt_hbm.at[pl.ds(tile_base, _K)], filt, semf).wait()

    def gather_issue(colv, rows, semg):
        pltpu.async_copy(dst_hbm.at[colv], rows, semg)

    def gather_wait(colv, rows, semg):
        pltpu.make_async_copy(dst_hbm.at[colv], rows, semg).wait()

    def mul_scatter(rows, filt, rowv):
        def _mul(k, _):
            for j in range(_H // 16):
                s = pl.ds(j * 16, 16)
                rows[k, s] = rows[k, s] * filt[k, s]
            return 0

        lax.fori_loop(0, _K, _mul, 0)
        pltpu.sync_copy(rows, acc.at[rowv], add=True)

    idx_copy(0, colv_a, rowv_a, semi_a)
    filt_copy(0, filt_a, semf_a)
    idx_copy(1, colv_b, rowv_b, semi_b)
    filt_copy(1, filt_b, semf_b)
    idx_wait(colv_a, rowv_a, semi_a)
    gather_issue(colv_a, rows_a, semg_a)

    def _pair(i, _):
        c0 = 2 * i
        idx_wait(colv_b, rowv_b, semi_b)
        gather_issue(colv_b, rows_b, semg_b)

        gather_wait(colv_a, rows_a, semg_a)
        filt_wait(filt_a, semf_a)
        mul_scatter(rows_a, filt_a, rowv_a)

        @pl.when(c0 + 2 < _NCHUNK)
        def _():
            idx_copy(c0 + 2, colv_a, rowv_a, semi_a)
            filt_copy(c0 + 2, filt_a, semf_a)

        gather_wait(colv_b, rows_b, semg_b)
        filt_wait(filt_b, semf_b)
        mul_scatter(rows_b, filt_b, rowv_b)

        @pl.when(c0 + 3 < _NCHUNK)
        def _():
            idx_copy(c0 + 3, colv_b, rowv_b, semi_b)
            filt_copy(c0 + 3, filt_b, semf_b)

        @pl.when(c0 + 2 < _NCHUNK)
        def _():
            idx_wait(colv_a, rowv_a, semi_a)
            gather_issue(colv_a, rows_a, semg_a)

        return 0

    lax.fori_loop(0, _NCHUNK // 2, _pair, 0)
    gather_wait(colv_a, rows_a, semg_a)
    filt_wait(filt_a, semf_a)
    mul_scatter(rows_a, filt_a, rowv_a)
    plsc.subcore_barrier()

    pltpu.sync_copy(acc.at[pl.ds(sid * _WPT, _WPT)],
                    out_hbm.at[cid, pl.ds(sid * _WPT, _WPT)])

    @pl.when(sid == 0)
    def _():
        pltpu.sync_copy(acc.at[pl.ds(_NS * _WPT, _TAIL)],
                        out_hbm.at[cid, pl.ds(_NS * _WPT, _TAIL)])


def _sc_gather_scatter(dst, filt, col, row):
    mesh = plsc.VectorSubcoreMesh(core_axis_name="c", subcore_axis_name="s")
    f = pl.kernel(
        _sc_body,
        mesh=mesh,
        out_type=jax.ShapeDtypeStruct((_NC, _N, _H), jnp.float32),
        scratch_types=[
            pltpu.VMEM((_K,), jnp.int32),
            pltpu.VMEM((_K,), jnp.int32),
            pltpu.VMEM((_K,), jnp.int32),
            pltpu.VMEM((_K,), jnp.int32),
            pltpu.VMEM((_K, _H), jnp.float32),
            pltpu.VMEM((_K, _H), jnp.float32),
            pltpu.VMEM((_K, _H), jnp.float32),
            pltpu.VMEM((_K, _H), jnp.float32),
            pltpu.VMEM((_ZB, _H), jnp.float32),
            pltpu.VMEM_SHARED((_N, _H), jnp.float32),
            pltpu.SemaphoreType.DMA,
            pltpu.SemaphoreType.DMA,
            pltpu.SemaphoreType.DMA,
            pltpu.SemaphoreType.DMA,
            pltpu.SemaphoreType.DMA,
            pltpu.SemaphoreType.DMA,
        ],
    )
    return f(dst, filt, col, row)


def _res_block(v, w_a, b_a, w_b, b_b):
    h = _silu(jnp.dot(v, w_a[...], preferred_element_type=jnp.float32) + b_a[...])
    h = _silu(jnp.dot(h, w_b[...], preferred_element_type=jnp.float32) + b_b[...])
    return v + h


def _post_chain(p0_ref, p1_ref, c_ref, src_ref, xin_ref, dw):
    conv = (p0_ref[...] + p1_ref[...]) * c_ref[0, 0]
    conv = _res_block(conv, *dw[0:4])
    conv = _res_block(conv, *dw[4:8])
    out = src_ref[...] * conv
    out = _res_block(out, *dw[8:12])
    out = _res_block(out, *dw[12:16])
    return out + xin_ref[...]


def _post_dual_body(p0_ref, p1_ref, c_ref, src_ref, xin_ref, *refs):
    dw = refs[0:16]
    aw = refs[16:24]
    x2_ref, src2_ref, dst2_ref = refs[24:27]
    x2 = _post_chain(p0_ref, p1_ref, c_ref, src_ref, xin_ref, dw)
    x2_ref[...] = x2
    h = _silu(jnp.dot(x2, aw[0][...], preferred_element_type=jnp.float32) + aw[1][...])
    h = _silu(jnp.dot(h, aw[2][...], preferred_element_type=jnp.float32) + aw[3][...])
    src2_ref[...] = x2 + h
    h = _silu(jnp.dot(x2, aw[4][...], preferred_element_type=jnp.float32) + aw[5][...])
    h = _silu(jnp.dot(h, aw[6][...], preferred_element_type=jnp.float32) + aw[7][...])
    dst2_ref[...] = x2 + h


def _post_dual(p0, p1, c, src, xin, conv_mlp, out_mlp, emb0, emb1):
    ws = []
    for res in conv_mlp + out_mlp:
        for (w, b) in res:
            ws.append(w)
            ws.append(b.reshape(1, _H))
    for (w, b) in emb0 + emb1:
        ws.append(w)
        ws.append(b.reshape(1, _H))
    blk = pl.BlockSpec((_BN, _H), lambda i: (i, 0))
    wspec = pl.BlockSpec((_H, _H), lambda i: (0, 0))
    bspec = pl.BlockSpec((1, _H), lambda i: (0, 0))
    return pl.pallas_call(
        _post_dual_body,
        grid=(_N // _BN,),
        in_specs=[blk, blk, pl.BlockSpec((1, 1), lambda i: (0, 0)), blk, blk]
        + [wspec, bspec] * 12,
        out_specs=[blk] * 3,
        out_shape=[jax.ShapeDtypeStruct((_N, _H), jnp.float32)] * 3,
    )(p0, p1, c.reshape(1, 1), src, xin, *ws)


def _post_pool_body(p0_ref, p1_ref, c_ref, src_ref, xin_ref, bi_ref, *refs):
    dw = refs[0:16]
    pw = refs[16:28]
    g_ref = refs[28]
    i = pl.program_id(0)
    out = _post_chain(p0_ref, p1_ref, c_ref, src_ref, xin_ref, dw)

    @pl.when(i == 0)
    def _():
        g_ref[...] = jnp.zeros((_G, _H), jnp.float32)

    oh = (bi_ref[0] == lax.broadcasted_iota(jnp.int32, (_G, _BN), 0)).astype(jnp.float32)
    g_ref[...] += jnp.dot(oh, out, preferred_element_type=jnp.float32)

    @pl.when(i == _N // _BN - 1)
    def _():
        g = g_ref[...]
        g = _res_block(g, *pw[0:4])
        g = _res_block(g, *pw[4:8])
        g = _res_block(g, *pw[8:12])
        g_ref[...] = g


def _post_pool(p0, p1, c, src, xin, batch_index, conv_mlp, out_mlp, pool_params):
    ws = []
    for res in conv_mlp + out_mlp:
        for (w, b) in res:
            ws.append(w)
            ws.append(b.reshape(1, _H))
    for res in pool_params:
        for (w, b) in res:
            ws.append(w)
            ws.append(b.reshape(1, _H))
    bi3 = batch_index.reshape(_N // _BN, 1, _BN)
    blk = pl.BlockSpec((_BN, _H), lambda i: (i, 0))
    wspec = pl.BlockSpec((_H, _H), lambda i: (0, 0))
    bspec = pl.BlockSpec((1, _H), lambda i: (0, 0))
    return pl.pallas_call(
        _post_pool_body,
        grid=(_N // _BN,),
        in_specs=[blk, blk, pl.BlockSpec((1, 1), lambda i: (0, 0)), blk, blk,
                  pl.BlockSpec((1, 1, _BN), lambda i: (i, 0, 0))]
        + [wspec, bspec] * 14,
        out_specs=pl.BlockSpec((_G, _H), lambda i: (0, 0)),
        out_shape=jax.ShapeDtypeStruct((_G, _H), jnp.float32),
    )(p0, p1, c.reshape(1, 1), src, xin, bi3, *ws)


def kernel(scalar, ef, edge_index, C, conv_smooth, batch_index, params):
    row = edge_index[0]
    col = edge_index[1]
    p1, p2 = params["convs"]
    f1 = _filters(ef, conv_smooth, p1["e_lin"])
    src1, dst1 = _dual_residual(scalar, p1["emb0"], p1["emb1"])
    part1 = _sc_gather_scatter(dst1, f1, col, row)
    f2 = _filters(ef, conv_smooth, p2["e_lin"])
    x2, src2, dst2 = _post_dual(part1[0], part1[1], C, src1, scalar,
                                p1["conv_mlp"], p1["out_mlp"],
                                p2["emb0"], p2["emb1"])
    part2 = _sc_gather_scatter(dst2, f2, col, row)
    return _post_pool(part2[0], part2[1], C, src2, x2, batch_index,
                      p2["conv_mlp"], p2["out_mlp"], params["pool"])
